# baseline (speedup 1.0000x reference)
import functools

import jax
import jax.numpy as jnp
from jax import lax
from jax.experimental import pallas as pl
from jax.experimental.pallas import tpu as pltpu
from jax.experimental.pallas import tpu_sc as plsc

VOCAB = 1000
BATCH = 1024
SEQ = 50
NSPLIT = 4
BSPLIT = BATCH // NSPLIT
NUM_WORKERS = 32
BPW = BSPLIT // NUM_WORKERS

_mesh = plsc.VectorSubcoreMesh(core_axis_name="c", subcore_axis_name="s")


@functools.partial(
    pl.kernel,
    mesh=_mesh,
    out_type=jax.ShapeDtypeStruct((BSPLIT, SEQ, VOCAB), jnp.float32),
    scratch_types=[
        pltpu.VMEM((BPW, SEQ), jnp.int32),
        pltpu.VMEM((SEQ, VOCAB), jnp.float32),
        pltpu.VMEM((SEQ, VOCAB), jnp.float32),
        pltpu.SemaphoreType.DMA,
        pltpu.SemaphoreType.DMA,
    ],
    compiler_params=pltpu.CompilerParams(use_tc_tiling_on_sc=False),
)
def _gather_rows(table_hbm, idx_hbm, out_hbm, idx_v, rows0, rows1, sem0, sem1):
    wid = lax.axis_index("s") * 2 + lax.axis_index("c")
    base = wid * BPW

    pltpu.sync_copy(idx_hbm.at[pl.ds(base, BPW)], idx_v)

    def gather(b, rows_v, sem):
        return pltpu.async_copy(table_hbm.at[idx_v.at[b]], rows_v, sem)

    def gather_wait(b, rows_v, sem):
        pltpu.make_async_copy(table_hbm.at[idx_v.at[b]], rows_v, sem).wait()

    def put(b, rows_v):
        pltpu.sync_copy(rows_v, out_hbm.at[base + b])

    gather(0, rows0, sem0)

    @pl.loop(0, BPW // 2)
    def _(j):
        b0 = j * 2
        gather_wait(b0, rows0, sem0)
        gather(b0 + 1, rows1, sem1)
        put(b0, rows0)
        gather_wait(b0 + 1, rows1, sem1)

        @pl.when(j < BPW // 2 - 1)
        def _():
            gather(b0 + 2, rows0, sem0)

        put(b0 + 1, rows1)


def kernel(x, embedding_weight):
    idx = x.astype(jnp.int32)
    parts = [
        _gather_rows(embedding_weight, idx[k * BSPLIT:(k + 1) * BSPLIT])
        for k in range(NSPLIT)
    ]
    return jnp.concatenate(parts, axis=0)


# speedup vs baseline: 1.6113x; 1.6113x over previous
import functools

import jax
import jax.numpy as jnp
from jax import lax
from jax.experimental import pallas as pl
from jax.experimental.pallas import tpu as pltpu
from jax.experimental.pallas import tpu_sc as plsc

VOCAB = 1000
VOCAB_PAD = 1024
BATCH = 1024
SEQ = 50
NUM_WORKERS = 32
BPW = BATCH // NUM_WORKERS

_mesh = plsc.VectorSubcoreMesh(core_axis_name="c", subcore_axis_name="s")


@functools.partial(
    pl.kernel,
    mesh=_mesh,
    out_type=jax.ShapeDtypeStruct((BATCH, SEQ, 8, 128), jnp.float32),
    scratch_types=[
        pltpu.VMEM((BPW, SEQ), jnp.int32),
        pltpu.VMEM((SEQ, 8, 128), jnp.float32),
        pltpu.VMEM((SEQ, 8, 128), jnp.float32),
        pltpu.SemaphoreType.DMA,
        pltpu.SemaphoreType.DMA,
    ],
)
def _gather_rows(table_hbm, idx_hbm, out_hbm, idx_v, rows0, rows1, sem0, sem1):
    wid = lax.axis_index("s") * 2 + lax.axis_index("c")
    base = wid * BPW

    pltpu.sync_copy(idx_hbm.at[pl.ds(base, BPW)], idx_v)

    def gather(b, rows_v, sem):
        return pltpu.async_copy(table_hbm.at[idx_v.at[b]], rows_v, sem)

    def gather_wait(b, rows_v, sem):
        pltpu.make_async_copy(table_hbm.at[idx_v.at[b]], rows_v, sem).wait()

    def put(b, rows_v):
        pltpu.sync_copy(rows_v, out_hbm.at[base + b])

    gather(0, rows0, sem0)

    @pl.loop(0, BPW // 2)
    def _(j):
        b0 = j * 2
        gather_wait(b0, rows0, sem0)
        gather(b0 + 1, rows1, sem1)
        put(b0, rows0)
        gather_wait(b0 + 1, rows1, sem1)

        @pl.when(j < BPW // 2 - 1)
        def _():
            gather(b0 + 2, rows0, sem0)

        put(b0 + 1, rows1)


def kernel(x, embedding_weight):
    idx = x.astype(jnp.int32)
    table = jnp.pad(embedding_weight, ((0, 0), (0, VOCAB_PAD - VOCAB)))
    table = table.reshape(VOCAB, 8, 128)
    out = _gather_rows(table, idx)
    return out.reshape(BATCH, SEQ, VOCAB_PAD)[:, :, :VOCAB]
